# baseline (speedup 1.0000x reference)
import functools

import jax
import jax.numpy as jnp
import numpy as np
from jax import lax
from jax.experimental import pallas as pl
from jax.experimental.pallas import tpu as pltpu
from jax.experimental.pallas import tpu_sc as plsc

_K = 100
_KP = 128
_B = 512
_D = 512
_F_SCORE = 1.2447
_LN_BETA = float(np.log(0.999))
_NC, _NS, _L = 2, 16, 16
_NW = _NC * _NS
_NCH = _D // _L
_GBLK = 64

_mesh = plsc.VectorSubcoreMesh(
    core_axis_name="c", subcore_axis_name="s", num_cores=_NC, num_subcores=_NS)


def _wid():
    return lax.axis_index("c") * _NS + lax.axis_index("s")


def _lane_iota():
    return lax.broadcasted_iota(jnp.int32, (_L,), 0)


def _sdiv(a, b):
    va = jnp.zeros((_L,), jnp.float32) + a
    vb = jnp.zeros((_L,), jnp.float32) + b
    return (va / vb)[0]


def _bsum_body(logits_ref, lab_ref, out_ref):
    x = logits_ref[...]
    labv = lab_ref[...]
    m = jnp.max(x, axis=1, keepdims=True)
    e = jnp.exp(x - m)
    p = e / jnp.sum(e, axis=1, keepdims=True)
    log_p = jnp.maximum(jnp.log(p), -100.0)
    log_1mp = jnp.maximum(jnp.log(1.0 - p), -100.0)
    oh = lax.broadcasted_iota(jnp.int32, x.shape, 1) == labv
    row = (jnp.sum(jnp.where(oh, log_p - log_1mp, 0.0), axis=1, keepdims=True)
           + jnp.sum(log_1mp, axis=1, keepdims=True))
    out_ref[...] = -row


def _bsum_tc(logits, labels):
    out = pl.pallas_call(
        _bsum_body,
        out_shape=jax.ShapeDtypeStruct((_B, 1), jnp.float32),
    )(logits, labels.reshape(_B, 1))
    return out.reshape(_B)


def _pab_body(feat_hbm, lab_hbm, cnt_hbm, stats_out,
              lab_v, idxb_v, rows_v, accr, meanr, cnt_v, stat_v, sem):
    w = _wid()
    lane = _lane_iota()
    pltpu.sync_copy(lab_hbm, lab_v)
    pltpu.sync_copy(cnt_hbm, cnt_v.at[pl.ds(0, _K)])
    zeros16 = jnp.zeros((_L,), jnp.float32)
    izeros16 = jnp.zeros((_L,), jnp.int32)

    cls_idx = w + _NW * lax.rem(lane, 4)
    cnt4 = plsc.load_gather(cnt_v, [cls_idx])
    inv4 = 1.0 / cnt4

    for j in range(_B // _L + 1):
        idxb_v[pl.ds(j * _L, _L)] = izeros16

    sz2 = 0.0
    stat = jnp.zeros((_L,), jnp.float32)
    for r in range(4):
        my_c = w + _NW * r
        n_r = 0
        for c in range(_B // _L):
            labc = lab_v[pl.ds(c * _L, _L)]
            m = labc == my_c
            plsc.store_compressed(idxb_v.at[pl.ds(n_r, _L)], c * _L + lane,
                                  mask=m)
            n_r = n_r + plsc.all_reduce_population_count(m)[0]
        nblk = lax.div(n_r + (_GBLK - 1), _GBLK)

        for j in range(_NCH):
            accr[0, pl.ds(j * _L, _L)] = zeros16

        def blk1(g, carry):
            pltpu.async_copy(feat_hbm.at[idxb_v.at[pl.ds(g * _GBLK, _GBLK)]],
                             rows_v, sem).wait()
            lim = jnp.minimum(n_r - g * _GBLK, _GBLK)

            def row1(i, c2):
                for j in range(_NCH):
                    accr[0, pl.ds(j * _L, _L)] = (
                        accr[0, pl.ds(j * _L, _L)]
                        + rows_v[i, pl.ds(j * _L, _L)])
                return c2
            lax.fori_loop(0, lim, row1, 0)
            return carry
        lax.fori_loop(0, nblk, blk1, 0)

        inv_r = inv4[r]
        for j in range(_NCH):
            meanr[0, pl.ds(j * _L, _L)] = accr[0, pl.ds(j * _L, _L)] * inv_r

        def blk2(g, carry):
            @pl.when(nblk > 1)
            def _():
                pltpu.async_copy(
                    feat_hbm.at[idxb_v.at[pl.ds(g * _GBLK, _GBLK)]],
                    rows_v, sem).wait()
            lim = jnp.minimum(n_r - g * _GBLK, _GBLK)

            def row2(i, c2):
                zs, nz, s2 = c2
                acc = jnp.zeros((_L,), jnp.float32)
                for j in range(_NCH):
                    f = rows_v[i, pl.ds(j * _L, _L)]
                    mv = meanr[0, pl.ds(j * _L, _L)]
                    acc = acc + jnp.where(f != 0.0, jnp.abs(f - mv), 0.0)
                z = jnp.sum(acc)
                return (zs + z, nz + jnp.where(z != 0.0, 1.0, 0.0),
                        s2 + z * z)
            return lax.fori_loop(0, lim, row2, carry)
        zs_r, nz_r, sz2 = lax.fori_loop(0, nblk, blk2, (0.0, 0.0, sz2))
        stat = jnp.where(lane == r, zs_r, stat)
        stat = jnp.where(lane == 4 + r, nz_r, stat)

    stat = jnp.where(lane == 8, sz2, stat)
    stat_v[...] = stat
    pltpu.sync_copy(stat_v, stats_out.at[pl.ds(w * _L, _L)])


_phase_ab = functools.partial(
    pl.kernel,
    out_type=jax.ShapeDtypeStruct((_NW * _L,), jnp.float32),
    mesh=_mesh,
    compiler_params=pltpu.CompilerParams(needs_layout_passes=False),
    scratch_types=[
        pltpu.VMEM((_B,), jnp.int32),
        pltpu.VMEM((_B + _L,), jnp.int32),
        pltpu.VMEM((_GBLK, _D), jnp.float32),
        pltpu.VMEM((1, _D), jnp.float32),
        pltpu.VMEM((1, _D), jnp.float32),
        pltpu.VMEM((_KP,), jnp.float32),
        pltpu.VMEM((_L,), jnp.float32),
        pltpu.SemaphoreType.DMA,
    ],
)(_pab_body)


def _sqrt16(x):
    xi = lax.bitcast_convert_type(x, jnp.int32)
    yi = jnp.int32(0x5F3759DF) - lax.shift_right_logical(xi, 1)
    y = lax.bitcast_convert_type(yi, jnp.float32)
    for _ in range(4):
        y = y * (1.5 - 0.5 * x * y * y)
    return x * y


def _pc_body(stats_hbm, lab_hbm, cnt_hbm, bsum_hbm, loss_out,
             stats_v, lab_v, cnt_v, bsum_v, zsum_v, zim_v, nz_v, sb_v, w_v,
             loss_v):
    @pl.when(_wid() == 0)
    def _():
        pltpu.sync_copy(stats_hbm, stats_v)
        pltpu.sync_copy(lab_hbm, lab_v)
        pltpu.sync_copy(cnt_hbm, cnt_v.at[pl.ds(0, _K)])
        pltpu.sync_copy(bsum_hbm, bsum_v)
        lane = _lane_iota()

        t0 = plsc.load_gather(stats_v, [lane * _L + 8])
        t1 = plsc.load_gather(stats_v, [(lane + _L) * _L + 8])
        sz2 = jnp.sum(t0 + t1)

        zm_acc = jnp.zeros((_L,), jnp.float32)
        n_acc = jnp.zeros((_L,), jnp.float32)
        for q in range(_KP // _L):
            cls = lane + q * _L
            tile = lax.rem(cls, _NW)
            r = lax.shift_right_logical(cls, 5)
            zsum_c = plsc.load_gather(stats_v, [tile * _L + r])
            nz_c = plsc.load_gather(stats_v, [tile * _L + 4 + r])
            valid = cls < _K
            cnt_c = jnp.where(valid, cnt_v[pl.ds(q * _L, _L)], 1.0)
            zim_c = zsum_c / cnt_c
            zsum_v[pl.ds(q * _L, _L)] = zsum_c
            zim_v[pl.ds(q * _L, _L)] = zim_c
            nz_v[pl.ds(q * _L, _L)] = nz_c
            zm_acc = zm_acc + jnp.where(valid, zim_c, 0.0)
            n_acc = n_acc + jnp.where(valid, cnt_c, 0.0)
        z_mean = jnp.sum(zm_acc) * (1.0 / _K)
        n_tot = jnp.sum(n_acc)

        cross_acc = jnp.zeros((_L,), jnp.float32)
        for q in range(_KP // _L):
            zim_c = zim_v[pl.ds(q * _L, _L)]
            zsum_c = zsum_v[pl.ds(q * _L, _L)]
            nz_c = nz_v[pl.ds(q * _L, _L)]
            cross_acc = cross_acc + zim_c * (zim_c * nz_c - 2.0 * zsum_c)
        ssw = _sdiv(sz2 + jnp.sum(cross_acc), n_tot - float(_K))

        ssb_acc = jnp.zeros((_L,), jnp.float32)
        for q in range(_KP // _L):
            valid = (_lane_iota() + q * _L) < _K
            cnt_c = jnp.where(valid, cnt_v[pl.ds(q * _L, _L)], 1.0)
            dzm = zim_v[pl.ds(q * _L, _L)] - z_mean
            sbm = jnp.where(valid, dzm * dzm * cnt_c, 0.0)
            sb_v[pl.ds(q * _L, _L)] = sbm
            ssb_acc = ssb_acc + sbm
        ssb = jnp.sum(ssb_acc) * (1.0 / (_K - 1))

        a = z_mean * z_mean
        inv2a = _sdiv(1.0, 2.0 * a)
        ws_acc = jnp.zeros((_L,), jnp.float32)
        for q in range(_KP // _L):
            valid = (_lane_iota() + q * _L) < _K
            zsum_c = zsum_v[pl.ds(q * _L, _L)]
            cnt_c = jnp.where(valid, cnt_v[pl.ds(q * _L, _L)], 1.0)
            sb_c = sb_v[pl.ds(q * _L, _L)]
            cq = _F_SCORE * ssw * float(_K - 1) - (ssb * float(_K - 1) - sb_c)
            bq = -(2.0 * z_mean * zsum_c + cq)
            d2 = bq * bq - 4.0 * a * (zsum_c * zsum_c)
            dok = d2 >= 0.0
            dq = _sqrt16(jnp.maximum(d2, 0.0))
            n_lb = jnp.abs((-bq - dq) * inv2a)
            n_ub = jnp.abs((-bq + dq) * inv2a)
            c1 = jnp.logical_and(dok, cnt_c < n_lb)
            c2 = jnp.logical_and(dok, cnt_c > n_ub)
            t = jnp.where(c1, 1.0 / (n_lb - cnt_c),
                          jnp.where(c2, 1.0 / (cnt_c - n_ub), 1.0))
            beta = jnp.exp(_LN_BETA * t)
            en = 1.0 - jnp.exp(_LN_BETA * t * cnt_c)
            wr = (1.0 - beta) / en
            wrm = jnp.where(valid, wr, 0.0)
            w_v[pl.ds(q * _L, _L)] = wrm
            ws_acc = ws_acc + wrm
        wsum = jnp.sum(ws_acc)

        def dotc(c, acc):
            labc = lab_v[pl.ds(c * _L, _L)]
            wg = plsc.load_gather(w_v, [labc])
            return acc + wg * bsum_v[pl.ds(c * _L, _L)]
        dot_acc = lax.fori_loop(0, _B // _L, dotc,
                                jnp.zeros((_L,), jnp.float32))
        loss = jnp.sum(dot_acc) * _sdiv(float(_K), wsum) * (1.0 / (_B * _K))
        loss_v[...] = jnp.zeros((_L,), jnp.float32) + loss
        pltpu.sync_copy(loss_v, loss_out)


_phase_c = functools.partial(
    pl.kernel,
    out_type=jax.ShapeDtypeStruct((_L,), jnp.float32),
    mesh=_mesh,
    compiler_params=pltpu.CompilerParams(needs_layout_passes=False),
    scratch_types=[
        pltpu.VMEM((_NW * _L,), jnp.float32),
        pltpu.VMEM((_B,), jnp.int32),
        pltpu.VMEM((_KP,), jnp.float32),
        pltpu.VMEM((_B,), jnp.float32),
        pltpu.VMEM((_KP,), jnp.float32),
        pltpu.VMEM((_KP,), jnp.float32),
        pltpu.VMEM((_KP,), jnp.float32),
        pltpu.VMEM((_KP,), jnp.float32),
        pltpu.VMEM((_KP,), jnp.float32),
        pltpu.VMEM((_L,), jnp.float32),
    ],
)(_pc_body)


def kernel(logits, labels, features, sample_num_per_cls):
    labels = labels.astype(jnp.int32)
    bsum = _bsum_tc(logits, labels)
    stats = _phase_ab(features, labels, sample_num_per_cls)
    loss_vec = _phase_c(stats, labels, sample_num_per_cls, bsum)
    return loss_vec[0]


# speedup vs baseline: 6.3401x; 6.3401x over previous
import functools

import jax
import jax.numpy as jnp
import numpy as np
from jax import lax
from jax.experimental import pallas as pl
from jax.experimental.pallas import tpu as pltpu
from jax.experimental.pallas import tpu_sc as plsc

_K = 100
_KP = 128
_B = 512
_D = 512
_F_SCORE = 1.2447
_LN_BETA = float(np.log(0.999))
_NC, _NS, _L = 2, 16, 16
_NW = _NC * _NS
_BPW = _B // _NW
_NCH = _D // _L
_ST = 2 * _KP + _L

_mesh = plsc.VectorSubcoreMesh(
    core_axis_name="c", subcore_axis_name="s", num_cores=_NC, num_subcores=_NS)


def _wid():
    return lax.axis_index("c") * _NS + lax.axis_index("s")


def _lane_iota():
    return lax.broadcasted_iota(jnp.int32, (_L,), 0)


def _sdiv(a, b):
    va = jnp.zeros((_L,), jnp.float32) + a
    vb = jnp.zeros((_L,), jnp.float32) + b
    return (va / vb)[0]


def _tc_body(logits_ref, lab_ref, feat_ref, bsum_ref, s_ref):
    x = logits_ref[...]
    labv = lab_ref[...]
    m = jnp.max(x, axis=1, keepdims=True)
    e = jnp.exp(x - m)
    p = e / jnp.sum(e, axis=1, keepdims=True)
    log_p = jnp.maximum(jnp.log(p), -100.0)
    log_1mp = jnp.maximum(jnp.log(1.0 - p), -100.0)
    oh = lax.broadcasted_iota(jnp.int32, x.shape, 1) == labv
    row = (jnp.sum(jnp.where(oh, log_p - log_1mp, 0.0), axis=1, keepdims=True)
           + jnp.sum(log_1mp, axis=1, keepdims=True))
    bsum_ref[...] = -row
    ohp = (lax.broadcasted_iota(jnp.int32, (_B, _KP), 1) == labv
           ).astype(jnp.float32)
    s_ref[...] = lax.dot_general(
        ohp, feat_ref[...], (((0,), (0,)), ((), ())),
        preferred_element_type=jnp.float32,
        precision=lax.Precision.HIGHEST)


def _tc_stage(logits, labels, features):
    return pl.pallas_call(
        _tc_body,
        out_shape=(jax.ShapeDtypeStruct((_B, 1), jnp.float32),
                   jax.ShapeDtypeStruct((_KP, _D), jnp.float32)),
    )(logits, labels.reshape(_B, 1), features)


def _pd_body(feat_hbm, lab_hbm, s_hbm, cnt_hbm, stats_out,
             feat_v, idx_v, rows, cnt_v, stat_v, sem, zsum_sm, nz_sm):
    w = _wid()
    base = w * _BPW
    lane = _lane_iota()
    pltpu.sync_copy(lab_hbm.at[pl.ds(base, _BPW)], idx_v)
    pltpu.sync_copy(cnt_hbm, cnt_v.at[pl.ds(0, _K)])
    pltpu.sync_copy(feat_hbm.at[pl.ds(base, _BPW)], feat_v)
    pltpu.async_copy(s_hbm.at[idx_v], rows, sem).wait()
    idxreg = idx_v[...]
    cntreg = plsc.load_gather(cnt_v, [idxreg])
    invreg = 1.0 / cntreg

    def zb(c, carry):
        zsum_sm[c] = 0.0
        nz_sm[c] = 0.0
        return carry
    lax.fori_loop(0, _KP, zb, 0)

    sz2 = 0.0
    for i in range(_BPW):
        inv = invreg[i]
        acc = jnp.zeros((_L,), jnp.float32)
        for j in range(_NCH):
            f = feat_v[i, pl.ds(j * _L, _L)]
            mv = rows[i, pl.ds(j * _L, _L)] * inv
            acc = acc + jnp.where(f != 0.0, jnp.abs(f - mv), 0.0)
        z = jnp.sum(acc)
        lab = idxreg[i]
        zsum_sm[lab] = zsum_sm[lab] + z
        nz_sm[lab] = nz_sm[lab] + jnp.where(z != 0.0, 1.0, 0.0)
        sz2 = sz2 + z * z

    for q in range(_KP // _L):
        vz = jnp.zeros((_L,), jnp.float32)
        vn = jnp.zeros((_L,), jnp.float32)
        for t in range(_L):
            vz = jnp.where(lane == t, zsum_sm[q * _L + t], vz)
            vn = jnp.where(lane == t, nz_sm[q * _L + t], vn)
        stat_v[pl.ds(q * _L, _L)] = vz
        stat_v[pl.ds(_KP + q * _L, _L)] = vn
    stat_v[pl.ds(2 * _KP, _L)] = jnp.where(lane == 0, sz2, 0.0)
    pltpu.sync_copy(stat_v, stats_out.at[pl.ds(w * _ST, _ST)])


_phase_d = functools.partial(
    pl.kernel,
    out_type=jax.ShapeDtypeStruct((_NW * _ST,), jnp.float32),
    mesh=_mesh,
    compiler_params=pltpu.CompilerParams(needs_layout_passes=False),
    scratch_types=[
        pltpu.VMEM((_BPW, _D), jnp.float32),
        pltpu.VMEM((_BPW,), jnp.int32),
        pltpu.VMEM((_BPW, _D), jnp.float32),
        pltpu.VMEM((_KP,), jnp.float32),
        pltpu.VMEM((_ST,), jnp.float32),
        pltpu.SemaphoreType.DMA,
        pltpu.SMEM((_KP,), jnp.float32),
        pltpu.SMEM((_KP,), jnp.float32),
    ],
)(_pd_body)


def _sqrt16(x):
    xi = lax.bitcast_convert_type(x, jnp.int32)
    yi = jnp.int32(0x5F3759DF) - lax.shift_right_logical(xi, 1)
    y = lax.bitcast_convert_type(yi, jnp.float32)
    for _ in range(4):
        y = y * (1.5 - 0.5 * x * y * y)
    return x * y


def _pc_body(stats_hbm, lab_hbm, cnt_hbm, bsum_hbm, loss_out,
             stats_v, lab_v, cnt_v, bsum_v, zsum_v, zim_v, nz_v, sb_v, w_v,
             loss_v):
    @pl.when(_wid() == 0)
    def _():
        pltpu.sync_copy(stats_hbm, stats_v)
        pltpu.sync_copy(lab_hbm, lab_v)
        pltpu.sync_copy(cnt_hbm, cnt_v.at[pl.ds(0, _K)])
        pltpu.sync_copy(bsum_hbm, bsum_v)

        sz2_acc = jnp.zeros((_L,), jnp.float32)
        for t in range(_NW):
            sz2_acc = sz2_acc + stats_v[pl.ds(t * _ST + 2 * _KP, _L)]
        sz2 = sz2_acc[0]

        zm_acc = jnp.zeros((_L,), jnp.float32)
        n_acc = jnp.zeros((_L,), jnp.float32)
        for q in range(_KP // _L):
            zsum_c = jnp.zeros((_L,), jnp.float32)
            nz_c = jnp.zeros((_L,), jnp.float32)
            for t in range(_NW):
                zsum_c = zsum_c + stats_v[pl.ds(t * _ST + q * _L, _L)]
                nz_c = nz_c + stats_v[pl.ds(t * _ST + _KP + q * _L, _L)]
            valid = (_lane_iota() + q * _L) < _K
            cnt_c = jnp.where(valid, cnt_v[pl.ds(q * _L, _L)], 1.0)
            zim_c = zsum_c / cnt_c
            zsum_v[pl.ds(q * _L, _L)] = zsum_c
            zim_v[pl.ds(q * _L, _L)] = zim_c
            nz_v[pl.ds(q * _L, _L)] = nz_c
            zm_acc = zm_acc + jnp.where(valid, zim_c, 0.0)
            n_acc = n_acc + jnp.where(valid, cnt_c, 0.0)
        z_mean = jnp.sum(zm_acc) * (1.0 / _K)
        n_tot = jnp.sum(n_acc)

        cross_acc = jnp.zeros((_L,), jnp.float32)
        for q in range(_KP // _L):
            zim_c = zim_v[pl.ds(q * _L, _L)]
            zsum_c = zsum_v[pl.ds(q * _L, _L)]
            nz_c = nz_v[pl.ds(q * _L, _L)]
            cross_acc = cross_acc + zim_c * (zim_c * nz_c - 2.0 * zsum_c)
        ssw = _sdiv(sz2 + jnp.sum(cross_acc), n_tot - float(_K))

        ssb_acc = jnp.zeros((_L,), jnp.float32)
        for q in range(_KP // _L):
            valid = (_lane_iota() + q * _L) < _K
            cnt_c = jnp.where(valid, cnt_v[pl.ds(q * _L, _L)], 1.0)
            dzm = zim_v[pl.ds(q * _L, _L)] - z_mean
            sbm = jnp.where(valid, dzm * dzm * cnt_c, 0.0)
            sb_v[pl.ds(q * _L, _L)] = sbm
            ssb_acc = ssb_acc + sbm
        ssb = jnp.sum(ssb_acc) * (1.0 / (_K - 1))

        a = z_mean * z_mean
        inv2a = _sdiv(1.0, 2.0 * a)
        ws_acc = jnp.zeros((_L,), jnp.float32)
        for q in range(_KP // _L):
            valid = (_lane_iota() + q * _L) < _K
            zsum_c = zsum_v[pl.ds(q * _L, _L)]
            cnt_c = jnp.where(valid, cnt_v[pl.ds(q * _L, _L)], 1.0)
            sb_c = sb_v[pl.ds(q * _L, _L)]
            cq = _F_SCORE * ssw * float(_K - 1) - (ssb * float(_K - 1) - sb_c)
            bq = -(2.0 * z_mean * zsum_c + cq)
            d2 = bq * bq - 4.0 * a * (zsum_c * zsum_c)
            dok = d2 >= 0.0
            dq = _sqrt16(jnp.maximum(d2, 0.0))
            n_lb = jnp.abs((-bq - dq) * inv2a)
            n_ub = jnp.abs((-bq + dq) * inv2a)
            c1 = jnp.logical_and(dok, cnt_c < n_lb)
            c2 = jnp.logical_and(dok, cnt_c > n_ub)
            t = jnp.where(c1, 1.0 / (n_lb - cnt_c),
                          jnp.where(c2, 1.0 / (cnt_c - n_ub), 1.0))
            beta = jnp.exp(_LN_BETA * t)
            en = 1.0 - jnp.exp(_LN_BETA * t * cnt_c)
            wr = (1.0 - beta) / en
            wrm = jnp.where(valid, wr, 0.0)
            w_v[pl.ds(q * _L, _L)] = wrm
            ws_acc = ws_acc + wrm
        wsum = jnp.sum(ws_acc)

        def dotc(c, acc):
            labc = lab_v[pl.ds(c * _L, _L)]
            wg = plsc.load_gather(w_v, [labc])
            return acc + wg * bsum_v[pl.ds(c * _L, _L)]
        dot_acc = lax.fori_loop(0, _B // _L, dotc,
                                jnp.zeros((_L,), jnp.float32))
        loss = jnp.sum(dot_acc) * _sdiv(float(_K), wsum) * (1.0 / (_B * _K))
        loss_v[...] = jnp.zeros((_L,), jnp.float32) + loss
        pltpu.sync_copy(loss_v, loss_out)


_phase_c = functools.partial(
    pl.kernel,
    out_type=jax.ShapeDtypeStruct((_L,), jnp.float32),
    mesh=_mesh,
    compiler_params=pltpu.CompilerParams(needs_layout_passes=False),
    scratch_types=[
        pltpu.VMEM((_NW * _ST,), jnp.float32),
        pltpu.VMEM((_B,), jnp.int32),
        pltpu.VMEM((_KP,), jnp.float32),
        pltpu.VMEM((_B,), jnp.float32),
        pltpu.VMEM((_KP,), jnp.float32),
        pltpu.VMEM((_KP,), jnp.float32),
        pltpu.VMEM((_KP,), jnp.float32),
        pltpu.VMEM((_KP,), jnp.float32),
        pltpu.VMEM((_KP,), jnp.float32),
        pltpu.VMEM((_L,), jnp.float32),
    ],
)(_pc_body)


def kernel(logits, labels, features, sample_num_per_cls):
    labels = labels.astype(jnp.int32)
    bsum, s_tab = _tc_stage(logits, labels, features)
    stats = _phase_d(features, labels, s_tab, sample_num_per_cls)
    loss_vec = _phase_c(stats, labels, sample_num_per_cls,
                        bsum.reshape(_B))
    return loss_vec[0]


# speedup vs baseline: 6.7040x; 1.0574x over previous
import functools

import jax
import jax.numpy as jnp
import numpy as np
from jax import lax
from jax.experimental import pallas as pl
from jax.experimental.pallas import tpu as pltpu
from jax.experimental.pallas import tpu_sc as plsc

_K = 100
_KP = 128
_B = 512
_D = 512
_F_SCORE = 1.2447
_LN_BETA = float(np.log(0.999))
_NC, _NS, _L = 2, 16, 16
_NW = _NC * _NS
_BPW = _B // _NW
_NCH = _D // _L
_ST = 2 * _KP + _L

_mesh = plsc.VectorSubcoreMesh(
    core_axis_name="c", subcore_axis_name="s", num_cores=_NC, num_subcores=_NS)


def _wid():
    return lax.axis_index("c") * _NS + lax.axis_index("s")


def _lane_iota():
    return lax.broadcasted_iota(jnp.int32, (_L,), 0)


def _sdiv(a, b):
    va = jnp.zeros((_L,), jnp.float32) + a
    vb = jnp.zeros((_L,), jnp.float32) + b
    return (va / vb)[0]


def _tc_s_body(lab_ref, feat_ref, s_ref):
    labv = lab_ref[...]
    ohp = (lax.broadcasted_iota(jnp.int32, (_B, _KP), 1) == labv
           ).astype(jnp.float32)
    s_ref[...] = lax.dot_general(
        ohp, feat_ref[...], (((0,), (0,)), ((), ())),
        preferred_element_type=jnp.float32,
        precision=lax.Precision.HIGHEST)


def _tc_s(labels, features):
    return pl.pallas_call(
        _tc_s_body,
        out_shape=jax.ShapeDtypeStruct((_KP, _D), jnp.float32),
    )(labels.reshape(_B, 1), features)


def _tc_bsum_body(logits_ref, lab_ref, bsum_ref):
    x = logits_ref[...]
    labv = lab_ref[...]
    m = jnp.max(x, axis=1, keepdims=True)
    e = jnp.exp(x - m)
    p = e / jnp.sum(e, axis=1, keepdims=True)
    log_p = jnp.maximum(jnp.log(p), -100.0)
    log_1mp = jnp.maximum(jnp.log(1.0 - p), -100.0)
    oh = lax.broadcasted_iota(jnp.int32, x.shape, 1) == labv
    row = (jnp.sum(jnp.where(oh, log_p - log_1mp, 0.0), axis=1, keepdims=True)
           + jnp.sum(log_1mp, axis=1, keepdims=True))
    bsum_ref[...] = -row


def _tc_bsum(logits, labels):
    out = pl.pallas_call(
        _tc_bsum_body,
        out_shape=jax.ShapeDtypeStruct((_B, 1), jnp.float32),
    )(logits, labels.reshape(_B, 1))
    return out.reshape(_B)


def _pd_body(feat_hbm, lab_hbm, s_hbm, cnt_hbm, stats_out,
             feat_v, idx_v, rows, cnt_v, stat_v, sem, zsum_sm, nz_sm):
    w = _wid()
    base = w * _BPW
    lane = _lane_iota()
    pltpu.sync_copy(lab_hbm.at[pl.ds(base, _BPW)], idx_v)
    pltpu.sync_copy(cnt_hbm, cnt_v.at[pl.ds(0, _K)])
    gat = pltpu.async_copy(s_hbm.at[idx_v], rows, sem)
    pltpu.sync_copy(feat_hbm.at[pl.ds(base, _BPW)], feat_v)
    idxreg = idx_v[...]
    cntreg = plsc.load_gather(cnt_v, [idxreg])
    invreg = 1.0 / cntreg

    def zb(c, carry):
        zsum_sm[c] = 0.0
        nz_sm[c] = 0.0
        return carry
    lax.fori_loop(0, _KP, zb, 0)
    gat.wait()

    sz2 = 0.0
    for i in range(_BPW):
        inv = invreg[i]
        acc = jnp.zeros((_L,), jnp.float32)
        for j in range(_NCH):
            f = feat_v[i, pl.ds(j * _L, _L)]
            mv = rows[i, pl.ds(j * _L, _L)] * inv
            acc = acc + jnp.where(f != 0.0, jnp.abs(f - mv), 0.0)
        z = jnp.sum(acc)
        lab = idxreg[i]
        zsum_sm[lab] = zsum_sm[lab] + z
        nz_sm[lab] = nz_sm[lab] + jnp.where(z != 0.0, 1.0, 0.0)
        sz2 = sz2 + z * z

    for q in range(_KP // _L):
        vz = jnp.zeros((_L,), jnp.float32)
        vn = jnp.zeros((_L,), jnp.float32)
        for t in range(_L):
            vz = jnp.where(lane == t, zsum_sm[q * _L + t], vz)
            vn = jnp.where(lane == t, nz_sm[q * _L + t], vn)
        stat_v[pl.ds(q * _L, _L)] = vz
        stat_v[pl.ds(_KP + q * _L, _L)] = vn
    stat_v[pl.ds(2 * _KP, _L)] = jnp.where(lane == 0, sz2, 0.0)
    pltpu.sync_copy(stat_v, stats_out.at[pl.ds(w * _ST, _ST)])


_phase_d = functools.partial(
    pl.kernel,
    out_type=jax.ShapeDtypeStruct((_NW * _ST,), jnp.float32),
    mesh=_mesh,
    compiler_params=pltpu.CompilerParams(needs_layout_passes=False),
    scratch_types=[
        pltpu.VMEM((_BPW, _D), jnp.float32),
        pltpu.VMEM((_BPW,), jnp.int32),
        pltpu.VMEM((_BPW, _D), jnp.float32),
        pltpu.VMEM((_KP,), jnp.float32),
        pltpu.VMEM((_ST,), jnp.float32),
        pltpu.SemaphoreType.DMA,
        pltpu.SMEM((_KP,), jnp.float32),
        pltpu.SMEM((_KP,), jnp.float32),
    ],
)(_pd_body)


def _sqrt16(x):
    xi = lax.bitcast_convert_type(x, jnp.int32)
    yi = jnp.int32(0x5F3759DF) - lax.shift_right_logical(xi, 1)
    y = lax.bitcast_convert_type(yi, jnp.float32)
    for _ in range(4):
        y = y * (1.5 - 0.5 * x * y * y)
    return x * y


def _pc_body(stats_hbm, lab_hbm, cnt_hbm, bsum_hbm, loss_out,
             stats_v, lab_v, cnt_v, bsum_v, w_v, loss_v):
    @pl.when(_wid() == 0)
    def _():
        pltpu.sync_copy(stats_hbm, stats_v)
        pltpu.sync_copy(lab_hbm, lab_v)
        pltpu.sync_copy(cnt_hbm, cnt_v.at[pl.ds(0, _K)])
        pltpu.sync_copy(bsum_hbm, bsum_v)

        sz2_acc = jnp.zeros((_L,), jnp.float32)
        for t in range(_NW):
            sz2_acc = sz2_acc + stats_v[pl.ds(t * _ST + 2 * _KP, _L)]
        sz2 = sz2_acc[0]

        nq = _KP // _L
        zsum_r, nz_r, zim_r, cnt_r, valid_r = [], [], [], [], []
        zm_acc = jnp.zeros((_L,), jnp.float32)
        n_acc = jnp.zeros((_L,), jnp.float32)
        for q in range(nq):
            zsum_c = jnp.zeros((_L,), jnp.float32)
            nz_c = jnp.zeros((_L,), jnp.float32)
            for t in range(_NW):
                zsum_c = zsum_c + stats_v[pl.ds(t * _ST + q * _L, _L)]
                nz_c = nz_c + stats_v[pl.ds(t * _ST + _KP + q * _L, _L)]
            valid = (_lane_iota() + q * _L) < _K
            cnt_c = jnp.where(valid, cnt_v[pl.ds(q * _L, _L)], 1.0)
            zim_c = zsum_c / cnt_c
            zsum_r.append(zsum_c)
            nz_r.append(nz_c)
            zim_r.append(zim_c)
            cnt_r.append(cnt_c)
            valid_r.append(valid)
            zm_acc = zm_acc + jnp.where(valid, zim_c, 0.0)
            n_acc = n_acc + jnp.where(valid, cnt_c, 0.0)
        z_mean = jnp.sum(zm_acc) * (1.0 / _K)
        n_tot = jnp.sum(n_acc)

        cross_acc = jnp.zeros((_L,), jnp.float32)
        for q in range(nq):
            cross_acc = cross_acc + zim_r[q] * (zim_r[q] * nz_r[q]
                                                - 2.0 * zsum_r[q])
        ssw = _sdiv(sz2 + jnp.sum(cross_acc), n_tot - float(_K))

        sb_r = []
        ssb_acc = jnp.zeros((_L,), jnp.float32)
        for q in range(nq):
            dzm = zim_r[q] - z_mean
            sbm = jnp.where(valid_r[q], dzm * dzm * cnt_r[q], 0.0)
            sb_r.append(sbm)
            ssb_acc = ssb_acc + sbm
        ssb = jnp.sum(ssb_acc) * (1.0 / (_K - 1))

        a = z_mean * z_mean
        inv2a = _sdiv(1.0, 2.0 * a)
        ws_acc = jnp.zeros((_L,), jnp.float32)
        for q in range(nq):
            valid = valid_r[q]
            zsum_c = zsum_r[q]
            cnt_c = cnt_r[q]
            sb_c = sb_r[q]
            cq = _F_SCORE * ssw * float(_K - 1) - (ssb * float(_K - 1) - sb_c)
            bq = -(2.0 * z_mean * zsum_c + cq)
            d2 = bq * bq - 4.0 * a * (zsum_c * zsum_c)
            dok = d2 >= 0.0
            dq = _sqrt16(jnp.maximum(d2, 0.0))
            n_lb = jnp.abs((-bq - dq) * inv2a)
            n_ub = jnp.abs((-bq + dq) * inv2a)
            c1 = jnp.logical_and(dok, cnt_c < n_lb)
            c2 = jnp.logical_and(dok, cnt_c > n_ub)
            t = jnp.where(c1, 1.0 / (n_lb - cnt_c),
                          jnp.where(c2, 1.0 / (cnt_c - n_ub), 1.0))
            beta = jnp.exp(_LN_BETA * t)
            en = 1.0 - jnp.exp(_LN_BETA * t * cnt_c)
            wr = (1.0 - beta) / en
            wrm = jnp.where(valid, wr, 0.0)
            w_v[pl.ds(q * _L, _L)] = wrm
            ws_acc = ws_acc + wrm
        wsum = jnp.sum(ws_acc)

        def dotc(c, acc):
            labc = lab_v[pl.ds(c * _L, _L)]
            wg = plsc.load_gather(w_v, [labc])
            return acc + wg * bsum_v[pl.ds(c * _L, _L)]
        dot_acc = lax.fori_loop(0, _B // _L, dotc,
                                jnp.zeros((_L,), jnp.float32))
        loss = jnp.sum(dot_acc) * _sdiv(float(_K), wsum) * (1.0 / (_B * _K))
        loss_v[...] = jnp.zeros((_L,), jnp.float32) + loss
        pltpu.sync_copy(loss_v, loss_out)


_phase_c = functools.partial(
    pl.kernel,
    out_type=jax.ShapeDtypeStruct((_L,), jnp.float32),
    mesh=_mesh,
    compiler_params=pltpu.CompilerParams(needs_layout_passes=False),
    scratch_types=[
        pltpu.VMEM((_NW * _ST,), jnp.float32),
        pltpu.VMEM((_B,), jnp.int32),
        pltpu.VMEM((_KP,), jnp.float32),
        pltpu.VMEM((_B,), jnp.float32),
        pltpu.VMEM((_KP,), jnp.float32),
        pltpu.VMEM((_L,), jnp.float32),
    ],
)(_pc_body)


def kernel(logits, labels, features, sample_num_per_cls):
    labels = labels.astype(jnp.int32)
    s_tab = _tc_s(labels, features)
    stats = _phase_d(features, labels, s_tab, sample_num_per_cls)
    bsum = _tc_bsum(logits, labels)
    loss_vec = _phase_c(stats, labels, sample_num_per_cls, bsum)
    return loss_vec[0]


# speedup vs baseline: 7.4059x; 1.1047x over previous
import functools

import jax
import jax.numpy as jnp
import numpy as np
from jax import lax
from jax.experimental import pallas as pl
from jax.experimental.pallas import tpu as pltpu
from jax.experimental.pallas import tpu_sc as plsc

_K = 100
_KP = 128
_B = 512
_D = 512
_F_SCORE = 1.2447
_LN_BETA = float(np.log(0.999))
_NC, _NS, _L = 2, 16, 16
_NW = _NC * _NS
_BPW = _B // _NW
_NCH = _D // _L
_ST = 2 * _KP + _L

_mesh = plsc.VectorSubcoreMesh(
    core_axis_name="c", subcore_axis_name="s", num_cores=_NC, num_subcores=_NS)


def _wid():
    return lax.axis_index("c") * _NS + lax.axis_index("s")


def _lane_iota():
    return lax.broadcasted_iota(jnp.int32, (_L,), 0)


def _sqrt16(x):
    xi = lax.bitcast_convert_type(x, jnp.int32)
    yi = jnp.int32(0x5F3759DF) - lax.shift_right_logical(xi, 1)
    y = lax.bitcast_convert_type(yi, jnp.float32)
    for _ in range(4):
        y = y * (1.5 - 0.5 * x * y * y)
    return x * y


def _sdiv(a, b):
    va = jnp.zeros((_L,), jnp.float32) + a
    vb = jnp.zeros((_L,), jnp.float32) + b
    return (va / vb)[0]


def _tc_s_body(lab_ref, feat_ref, s_ref):
    labv = lab_ref[...]
    ohp = (lax.broadcasted_iota(jnp.int32, (_B, _KP), 1) == labv
           ).astype(jnp.float32)
    s_ref[...] = lax.dot_general(
        ohp, feat_ref[...], (((0,), (0,)), ((), ())),
        preferred_element_type=jnp.float32,
        precision=lax.Precision.HIGHEST)


def _tc_s(labels, features):
    return pl.pallas_call(
        _tc_s_body,
        out_shape=jax.ShapeDtypeStruct((_KP, _D), jnp.float32),
    )(labels.reshape(_B, 1), features)


def _tc_bsum_body(logits_ref, lab_ref, bsum_ref):
    x = logits_ref[...]
    labv = lab_ref[...]
    m = jnp.max(x, axis=1, keepdims=True)
    e = jnp.exp(x - m)
    p = e / jnp.sum(e, axis=1, keepdims=True)
    log_p = jnp.maximum(jnp.log(p), -100.0)
    log_1mp = jnp.maximum(jnp.log(1.0 - p), -100.0)
    oh = lax.broadcasted_iota(jnp.int32, x.shape, 1) == labv
    row = (jnp.sum(jnp.where(oh, log_p - log_1mp, 0.0), axis=1, keepdims=True)
           + jnp.sum(log_1mp, axis=1, keepdims=True))
    bsum_ref[...] = -row


def _tc_bsum(logits, labels):
    out = pl.pallas_call(
        _tc_bsum_body,
        out_shape=jax.ShapeDtypeStruct((_B, 1), jnp.float32),
    )(logits, labels.reshape(_B, 1))
    return out.reshape(_B)


_SPT = _B // _NS


def _pdc_body(feat_hbm, lab_hbm, s_hbm, cnt_hbm, bsum_hbm,
              stats_out, loss_out,
              feat_v, idx_v, rows, cnt_v, inv_v, stat_v,
              stats_v, lab_v, bsum_v, w_v, loss_v, sem,
              zsum_sm, nz_sm):
    cid = lax.axis_index("c")
    sid = lax.axis_index("s")
    lane = _lane_iota()

    @pl.when(cid == 0)
    def _():
        base = sid * _SPT
        pltpu.sync_copy(lab_hbm.at[pl.ds(base, _SPT)], idx_v.at[pl.ds(0, _SPT)])
        pltpu.sync_copy(cnt_hbm, cnt_v.at[pl.ds(0, _K)])
        gat = pltpu.async_copy(s_hbm.at[idx_v.at[pl.ds(0, _SPT)]], rows, sem)
        pltpu.sync_copy(feat_hbm.at[pl.ds(base, _SPT)], feat_v)
        for h in range(_SPT // _L):
            idxreg = idx_v[pl.ds(h * _L, _L)]
            inv_v[pl.ds(h * _L, _L)] = 1.0 / plsc.load_gather(cnt_v, [idxreg])

        def zb(c, carry):
            zsum_sm[c] = 0.0
            nz_sm[c] = 0.0
            return carry
        lax.fori_loop(0, _KP, zb, 0)
        gat.wait()

        def sample(i, sz2):
            inv = inv_v[pl.ds(i, _L)][0]
            lab = idx_v[pl.ds(i, _L)][0]
            acc = jnp.zeros((_L,), jnp.float32)
            for j in range(_NCH):
                f = feat_v[i, pl.ds(j * _L, _L)]
                mv = rows[i, pl.ds(j * _L, _L)] * inv
                acc = acc + jnp.where(f != 0.0, jnp.abs(f - mv), 0.0)
            z = jnp.sum(acc)
            zsum_sm[lab] = zsum_sm[lab] + z
            nz_sm[lab] = nz_sm[lab] + jnp.where(z != 0.0, 1.0, 0.0)
            return sz2 + z * z
        sz2 = lax.fori_loop(0, _SPT, sample, 0.0)

        for q in range(_KP // _L):
            vz = jnp.zeros((_L,), jnp.float32)
            vn = jnp.zeros((_L,), jnp.float32)
            for t in range(_L):
                vz = jnp.where(lane == t, zsum_sm[q * _L + t], vz)
                vn = jnp.where(lane == t, nz_sm[q * _L + t], vn)
            stat_v[pl.ds(q * _L, _L)] = vz
            stat_v[pl.ds(_KP + q * _L, _L)] = vn
        stat_v[pl.ds(2 * _KP, _L)] = jnp.where(lane == 0, sz2, 0.0)
        pltpu.sync_copy(stat_v, stats_out.at[pl.ds(sid * _ST, _ST)])

    plsc.subcore_barrier()

    @pl.when(jnp.logical_and(cid == 0, sid == 0))
    def _():
        pltpu.sync_copy(stats_out, stats_v)
        pltpu.sync_copy(lab_hbm, lab_v)
        pltpu.sync_copy(bsum_hbm, bsum_v)

        sz2_acc = jnp.zeros((_L,), jnp.float32)
        for t in range(_NS):
            sz2_acc = sz2_acc + stats_v[pl.ds(t * _ST + 2 * _KP, _L)]
        sz2 = sz2_acc[0]

        nq = _KP // _L
        zsum_r, nz_r, zim_r, cnt_r, valid_r = [], [], [], [], []
        zm_acc = jnp.zeros((_L,), jnp.float32)
        n_acc = jnp.zeros((_L,), jnp.float32)
        for q in range(nq):
            zsum_c = jnp.zeros((_L,), jnp.float32)
            nz_c = jnp.zeros((_L,), jnp.float32)
            for t in range(_NS):
                zsum_c = zsum_c + stats_v[pl.ds(t * _ST + q * _L, _L)]
                nz_c = nz_c + stats_v[pl.ds(t * _ST + _KP + q * _L, _L)]
            valid = (_lane_iota() + q * _L) < _K
            cnt_c = jnp.where(valid, cnt_v[pl.ds(q * _L, _L)], 1.0)
            zim_c = zsum_c / cnt_c
            zsum_r.append(zsum_c)
            nz_r.append(nz_c)
            zim_r.append(zim_c)
            cnt_r.append(cnt_c)
            valid_r.append(valid)
            zm_acc = zm_acc + jnp.where(valid, zim_c, 0.0)
            n_acc = n_acc + jnp.where(valid, cnt_c, 0.0)
        z_mean = jnp.sum(zm_acc) * (1.0 / _K)
        n_tot = jnp.sum(n_acc)

        cross_acc = jnp.zeros((_L,), jnp.float32)
        for q in range(nq):
            cross_acc = cross_acc + zim_r[q] * (zim_r[q] * nz_r[q]
                                                - 2.0 * zsum_r[q])
        ssw = _sdiv(sz2 + jnp.sum(cross_acc), n_tot - float(_K))

        sb_r = []
        ssb_acc = jnp.zeros((_L,), jnp.float32)
        for q in range(nq):
            dzm = zim_r[q] - z_mean
            sbm = jnp.where(valid_r[q], dzm * dzm * cnt_r[q], 0.0)
            sb_r.append(sbm)
            ssb_acc = ssb_acc + sbm
        ssb = jnp.sum(ssb_acc) * (1.0 / (_K - 1))

        a = z_mean * z_mean
        inv2a = _sdiv(1.0, 2.0 * a)
        ws_acc = jnp.zeros((_L,), jnp.float32)
        for q in range(nq):
            cq = _F_SCORE * ssw * float(_K - 1) - (ssb * float(_K - 1)
                                                   - sb_r[q])
            bq = -(2.0 * z_mean * zsum_r[q] + cq)
            d2 = bq * bq - 4.0 * a * (zsum_r[q] * zsum_r[q])
            dok = d2 >= 0.0
            dq = _sqrt16(jnp.maximum(d2, 0.0))
            n_lb = jnp.abs((-bq - dq) * inv2a)
            n_ub = jnp.abs((-bq + dq) * inv2a)
            c1 = jnp.logical_and(dok, cnt_r[q] < n_lb)
            c2 = jnp.logical_and(dok, cnt_r[q] > n_ub)
            t = jnp.where(c1, 1.0 / (n_lb - cnt_r[q]),
                          jnp.where(c2, 1.0 / (cnt_r[q] - n_ub), 1.0))
            beta = jnp.exp(_LN_BETA * t)
            en = 1.0 - jnp.exp(_LN_BETA * t * cnt_r[q])
            wr = (1.0 - beta) / en
            wrm = jnp.where(valid_r[q], wr, 0.0)
            w_v[pl.ds(q * _L, _L)] = wrm
            ws_acc = ws_acc + wrm
        wsum = jnp.sum(ws_acc)

        def dotc(c, acc):
            labc = lab_v[pl.ds(c * _L, _L)]
            wg = plsc.load_gather(w_v, [labc])
            return acc + wg * bsum_v[pl.ds(c * _L, _L)]
        dot_acc = lax.fori_loop(0, _B // _L, dotc,
                                jnp.zeros((_L,), jnp.float32))
        loss = jnp.sum(dot_acc) * _sdiv(float(_K), wsum) * (1.0 / (_B * _K))
        loss_v[...] = jnp.zeros((_L,), jnp.float32) + loss
        pltpu.sync_copy(loss_v, loss_out)


_phase_dc = functools.partial(
    pl.kernel,
    out_type=(jax.ShapeDtypeStruct((_NS * _ST,), jnp.float32),
              jax.ShapeDtypeStruct((_L,), jnp.float32)),
    mesh=_mesh,
    compiler_params=pltpu.CompilerParams(needs_layout_passes=False),
    scratch_types=[
        pltpu.VMEM((_SPT, _D), jnp.float32),
        pltpu.VMEM((_SPT + _L,), jnp.int32),
        pltpu.VMEM((_SPT, _D), jnp.float32),
        pltpu.VMEM((_KP,), jnp.float32),
        pltpu.VMEM((_SPT + _L,), jnp.float32),
        pltpu.VMEM((_ST,), jnp.float32),
        pltpu.VMEM((_NS * _ST,), jnp.float32),
        pltpu.VMEM((_B,), jnp.int32),
        pltpu.VMEM((_B,), jnp.float32),
        pltpu.VMEM((_KP,), jnp.float32),
        pltpu.VMEM((_L,), jnp.float32),
        pltpu.SemaphoreType.DMA,
        pltpu.SMEM((_KP,), jnp.float32),
        pltpu.SMEM((_KP,), jnp.float32),
    ],
)(_pdc_body)


def kernel(logits, labels, features, sample_num_per_cls):
    labels = labels.astype(jnp.int32)
    s_tab = _tc_s(labels, features)
    bsum = _tc_bsum(logits, labels)
    _, loss_vec = _phase_dc(features, labels, s_tab, sample_num_per_cls,
                            bsum)
    return loss_vec[0]


# speedup vs baseline: 7.5628x; 1.0212x over previous
import functools

import jax
import jax.numpy as jnp
import numpy as np
from jax import lax
from jax.experimental import pallas as pl
from jax.experimental.pallas import tpu as pltpu
from jax.experimental.pallas import tpu_sc as plsc

_K = 100
_KP = 128
_B = 512
_D = 512
_F_SCORE = 1.2447
_LN_BETA = float(np.log(0.999))
_NC, _NS, _L = 2, 16, 16
_NW = _NC * _NS
_BPW = _B // _NW
_NCH = _D // _L
_ST = 2 * _KP + _L

_mesh = plsc.VectorSubcoreMesh(
    core_axis_name="c", subcore_axis_name="s", num_cores=_NC, num_subcores=_NS)


def _wid():
    return lax.axis_index("c") * _NS + lax.axis_index("s")


def _lane_iota():
    return lax.broadcasted_iota(jnp.int32, (_L,), 0)


def _sqrt16(x):
    xi = lax.bitcast_convert_type(x, jnp.int32)
    yi = jnp.int32(0x5F3759DF) - lax.shift_right_logical(xi, 1)
    y = lax.bitcast_convert_type(yi, jnp.float32)
    for _ in range(4):
        y = y * (1.5 - 0.5 * x * y * y)
    return x * y


def _sdiv(a, b):
    va = jnp.zeros((_L,), jnp.float32) + a
    vb = jnp.zeros((_L,), jnp.float32) + b
    return (va / vb)[0]


def _tc_body(logits_ref, lab_ref, feat_ref, bsum_ref, s_ref):
    x = logits_ref[...]
    labv = lab_ref[...]
    m = jnp.max(x, axis=1, keepdims=True)
    e = jnp.exp(x - m)
    p = e / jnp.sum(e, axis=1, keepdims=True)
    log_p = jnp.maximum(jnp.log(p), -100.0)
    log_1mp = jnp.maximum(jnp.log(1.0 - p), -100.0)
    oh = lax.broadcasted_iota(jnp.int32, x.shape, 1) == labv
    row = (jnp.sum(jnp.where(oh, log_p - log_1mp, 0.0), axis=1, keepdims=True)
           + jnp.sum(log_1mp, axis=1, keepdims=True))
    bsum_ref[...] = -row
    ohp = (lax.broadcasted_iota(jnp.int32, (_B, _KP), 1) == labv
           ).astype(jnp.float32)
    s_ref[...] = lax.dot_general(
        ohp, feat_ref[...], (((0,), (0,)), ((), ())),
        preferred_element_type=jnp.float32,
        precision=lax.Precision.HIGHEST)


def _tc_stage(logits, labels, features):
    return pl.pallas_call(
        _tc_body,
        out_shape=(jax.ShapeDtypeStruct((_B, 1), jnp.float32),
                   jax.ShapeDtypeStruct((_KP, _D), jnp.float32)),
    )(logits, labels.reshape(_B, 1), features)


_SPT = _B // _NS


def _pdc_body(feat_hbm, lab_hbm, s_hbm, cnt_hbm, bsum_hbm,
              stats_out, loss_out,
              feat_v, idx_v, rows, cnt_v, inv_v, stat_v,
              stats_v, lab_v, bsum_v, w_v, loss_v, sem,
              zsum_sm, nz_sm):
    cid = lax.axis_index("c")
    sid = lax.axis_index("s")
    lane = _lane_iota()

    @pl.when(cid == 0)
    def _():
        base = sid * _SPT
        pltpu.sync_copy(lab_hbm.at[pl.ds(base, _SPT)], idx_v.at[pl.ds(0, _SPT)])
        pltpu.sync_copy(cnt_hbm, cnt_v.at[pl.ds(0, _K)])
        gat = pltpu.async_copy(s_hbm.at[idx_v.at[pl.ds(0, _SPT)]], rows, sem)
        pltpu.sync_copy(feat_hbm.at[pl.ds(base, _SPT)], feat_v)
        for h in range(_SPT // _L):
            idxreg = idx_v[pl.ds(h * _L, _L)]
            inv_v[pl.ds(h * _L, _L)] = 1.0 / plsc.load_gather(cnt_v, [idxreg])

        def zb(c, carry):
            zsum_sm[c] = 0.0
            nz_sm[c] = 0.0
            return carry
        lax.fori_loop(0, _KP, zb, 0)
        gat.wait()

        def sample(i, sz2):
            inv = inv_v[pl.ds(i, _L)][0]
            lab = idx_v[pl.ds(i, _L)][0]
            acc = jnp.zeros((_L,), jnp.float32)
            for j in range(_NCH):
                f = feat_v[i, pl.ds(j * _L, _L)]
                mv = rows[i, pl.ds(j * _L, _L)] * inv
                acc = acc + jnp.where(f != 0.0, jnp.abs(f - mv), 0.0)
            z = jnp.sum(acc)
            zsum_sm[lab] = zsum_sm[lab] + z
            nz_sm[lab] = nz_sm[lab] + jnp.where(z != 0.0, 1.0, 0.0)
            return sz2 + z * z
        sz2 = lax.fori_loop(0, _SPT, sample, 0.0)

        for q in range(_KP // _L):
            vz = jnp.zeros((_L,), jnp.float32)
            vn = jnp.zeros((_L,), jnp.float32)
            for t in range(_L):
                vz = jnp.where(lane == t, zsum_sm[q * _L + t], vz)
                vn = jnp.where(lane == t, nz_sm[q * _L + t], vn)
            stat_v[pl.ds(q * _L, _L)] = vz
            stat_v[pl.ds(_KP + q * _L, _L)] = vn
        stat_v[pl.ds(2 * _KP, _L)] = jnp.where(lane == 0, sz2, 0.0)
        pltpu.sync_copy(stat_v, stats_out.at[pl.ds(sid * _ST, _ST)])

    plsc.subcore_barrier()

    @pl.when(jnp.logical_and(cid == 0, sid == 0))
    def _():
        pltpu.sync_copy(stats_out, stats_v)
        pltpu.sync_copy(lab_hbm, lab_v)
        pltpu.sync_copy(bsum_hbm, bsum_v)

        sz2_acc = jnp.zeros((_L,), jnp.float32)
        for t in range(_NS):
            sz2_acc = sz2_acc + stats_v[pl.ds(t * _ST + 2 * _KP, _L)]
        sz2 = sz2_acc[0]

        nq = _KP // _L
        zsum_r, nz_r, zim_r, cnt_r, valid_r = [], [], [], [], []
        zm_acc = jnp.zeros((_L,), jnp.float32)
        n_acc = jnp.zeros((_L,), jnp.float32)
        for q in range(nq):
            zsum_c = jnp.zeros((_L,), jnp.float32)
            nz_c = jnp.zeros((_L,), jnp.float32)
            for t in range(_NS):
                zsum_c = zsum_c + stats_v[pl.ds(t * _ST + q * _L, _L)]
                nz_c = nz_c + stats_v[pl.ds(t * _ST + _KP + q * _L, _L)]
            valid = (_lane_iota() + q * _L) < _K
            cnt_c = jnp.where(valid, cnt_v[pl.ds(q * _L, _L)], 1.0)
            zim_c = zsum_c / cnt_c
            zsum_r.append(zsum_c)
            nz_r.append(nz_c)
            zim_r.append(zim_c)
            cnt_r.append(cnt_c)
            valid_r.append(valid)
            zm_acc = zm_acc + jnp.where(valid, zim_c, 0.0)
            n_acc = n_acc + jnp.where(valid, cnt_c, 0.0)
        z_mean = jnp.sum(zm_acc) * (1.0 / _K)
        n_tot = jnp.sum(n_acc)

        cross_acc = jnp.zeros((_L,), jnp.float32)
        for q in range(nq):
            cross_acc = cross_acc + zim_r[q] * (zim_r[q] * nz_r[q]
                                                - 2.0 * zsum_r[q])
        ssw = _sdiv(sz2 + jnp.sum(cross_acc), n_tot - float(_K))

        sb_r = []
        ssb_acc = jnp.zeros((_L,), jnp.float32)
        for q in range(nq):
            dzm = zim_r[q] - z_mean
            sbm = jnp.where(valid_r[q], dzm * dzm * cnt_r[q], 0.0)
            sb_r.append(sbm)
            ssb_acc = ssb_acc + sbm
        ssb = jnp.sum(ssb_acc) * (1.0 / (_K - 1))

        a = z_mean * z_mean
        inv2a = _sdiv(1.0, 2.0 * a)
        ws_acc = jnp.zeros((_L,), jnp.float32)
        for q in range(nq):
            cq = _F_SCORE * ssw * float(_K - 1) - (ssb * float(_K - 1)
                                                   - sb_r[q])
            bq = -(2.0 * z_mean * zsum_r[q] + cq)
            d2 = bq * bq - 4.0 * a * (zsum_r[q] * zsum_r[q])
            dok = d2 >= 0.0
            dq = _sqrt16(jnp.maximum(d2, 0.0))
            n_lb = jnp.abs((-bq - dq) * inv2a)
            n_ub = jnp.abs((-bq + dq) * inv2a)
            c1 = jnp.logical_and(dok, cnt_r[q] < n_lb)
            c2 = jnp.logical_and(dok, cnt_r[q] > n_ub)
            t = jnp.where(c1, 1.0 / (n_lb - cnt_r[q]),
                          jnp.where(c2, 1.0 / (cnt_r[q] - n_ub), 1.0))
            beta = jnp.exp(_LN_BETA * t)
            en = 1.0 - jnp.exp(_LN_BETA * t * cnt_r[q])
            wr = (1.0 - beta) / en
            wrm = jnp.where(valid_r[q], wr, 0.0)
            w_v[pl.ds(q * _L, _L)] = wrm
            ws_acc = ws_acc + wrm
        wsum = jnp.sum(ws_acc)

        def dotc(c, acc):
            labc = lab_v[pl.ds(c * _L, _L)]
            wg = plsc.load_gather(w_v, [labc])
            return acc + wg * bsum_v[pl.ds(c * _L, _L)]
        dot_acc = lax.fori_loop(0, _B // _L, dotc,
                                jnp.zeros((_L,), jnp.float32))
        loss = jnp.sum(dot_acc) * _sdiv(float(_K), wsum) * (1.0 / (_B * _K))
        loss_v[...] = jnp.zeros((_L,), jnp.float32) + loss
        pltpu.sync_copy(loss_v, loss_out)


_phase_dc = functools.partial(
    pl.kernel,
    out_type=(jax.ShapeDtypeStruct((_NS * _ST,), jnp.float32),
              jax.ShapeDtypeStruct((_L,), jnp.float32)),
    mesh=_mesh,
    compiler_params=pltpu.CompilerParams(needs_layout_passes=False),
    scratch_types=[
        pltpu.VMEM((_SPT, _D), jnp.float32),
        pltpu.VMEM((_SPT + _L,), jnp.int32),
        pltpu.VMEM((_SPT, _D), jnp.float32),
        pltpu.VMEM((_KP,), jnp.float32),
        pltpu.VMEM((_SPT + _L,), jnp.float32),
        pltpu.VMEM((_ST,), jnp.float32),
        pltpu.VMEM((_NS * _ST,), jnp.float32),
        pltpu.VMEM((_B,), jnp.int32),
        pltpu.VMEM((_B,), jnp.float32),
        pltpu.VMEM((_KP,), jnp.float32),
        pltpu.VMEM((_L,), jnp.float32),
        pltpu.SemaphoreType.DMA,
        pltpu.SMEM((_KP,), jnp.float32),
        pltpu.SMEM((_KP,), jnp.float32),
    ],
)(_pdc_body)


def kernel(logits, labels, features, sample_num_per_cls):
    labels = labels.astype(jnp.int32)
    bsum, s_tab = _tc_stage(logits, labels, features)
    _, loss_vec = _phase_dc(features, labels, s_tab, sample_num_per_cls,
                            bsum.reshape(_B))
    return loss_vec[0]
